# Initial kernel scaffold; baseline (speedup 1.0000x reference)
import jax
import jax.numpy as jnp
from jax.experimental import pallas as pl


def kernel(x, edge_index, W1, b1, W2, b2, Wl, bl):
    raise NotImplementedError("write your pallas kernel here")


# speedup vs baseline: 17.2585x; 17.2585x over previous
import functools

import jax
import jax.numpy as jnp
from jax import lax
from jax.experimental import pallas as pl
from jax.experimental.pallas import tpu as pltpu
from jax.experimental.pallas import tpu_sc as plsc

N = 10000
E = 320000
D_IN = 128
D_H = 128
D_EMB = 64
D_OUT = 40

NC = 2
NS = 16
NW = NC * NS
CHUNK = 128
CPT = 79
EP = NW * CPT * CHUNK
NCH = EP // CHUNK
NACC = 10240
RPT = NACC // NS

_MESH = plsc.VectorSubcoreMesh(core_axis_name="c", subcore_axis_name="s")


def _deg_partials(dstp):
    @functools.partial(
        pl.kernel,
        out_type=jax.ShapeDtypeStruct((NC, NACC, 16), jnp.float32),
        mesh=_MESH,
        compiler_params=pltpu.CompilerParams(use_tc_tiling_on_sc=False),
        scratch_types=[
            pltpu.VMEM((2, CHUNK), jnp.int32),
            pltpu.VMEM((CHUNK, 16), jnp.float32),
            pltpu.VMEM((CHUNK, 16), jnp.float32),
            pltpu.VMEM_SHARED((NACC, 16), jnp.float32),
        ],
    )
    def degk(dst_hbm, out_hbm, dst_v, ones_v, zeros_v, acc_sh):
        c = lax.axis_index("c")
        s = lax.axis_index("s")

        @pl.loop(0, CHUNK)
        def _(r):
            ones_v[r, pl.ds(0, 16)] = jnp.ones((16,), jnp.float32)
            zeros_v[r, pl.ds(0, 16)] = jnp.zeros((16,), jnp.float32)

        @pl.loop(0, RPT // CHUNK)
        def _(j):
            pltpu.sync_copy(
                zeros_v, acc_sh.at[pl.ds(s * RPT + j * CHUNK, CHUNK)])

        plsc.subcore_barrier()
        base = (c * NS + s) * CPT

        @pl.loop(0, CPT)
        def _(k):
            pltpu.sync_copy(dst_hbm.at[base + k], dst_v.at[0])
            pltpu.sync_copy(ones_v, acc_sh.at[dst_v.at[0]], add=True)

        plsc.subcore_barrier()
        pltpu.sync_copy(acc_sh.at[pl.ds(s * RPT, RPT)],
                        out_hbm.at[c, pl.ds(s * RPT, RPT)])

    return degk(dstp)


def _segsum_partials(g, srcp, dstp, d):
    @functools.partial(
        pl.kernel,
        out_type=jax.ShapeDtypeStruct((NC, NACC, d), jnp.float32),
        mesh=_MESH,
        compiler_params=pltpu.CompilerParams(use_tc_tiling_on_sc=(d == 128)),
        scratch_types=[
            pltpu.VMEM((2, CHUNK), jnp.int32),
            pltpu.VMEM((2, CHUNK), jnp.int32),
            pltpu.VMEM((2, CHUNK, d), jnp.float32),
            pltpu.VMEM_SHARED((NACC, d), jnp.float32),
            pltpu.SemaphoreType.DMA,
        ],
    )
    def segk(g_hbm, src_hbm, dst_hbm, out_hbm, src_v, dst_v, rows_v, acc_sh,
             sem):
        c = lax.axis_index("c")
        s = lax.axis_index("s")

        @pl.loop(0, CHUNK)
        def _(r):
            for j in range(d // 16):
                rows_v[0, r, pl.ds(j * 16, 16)] = jnp.zeros((16,), jnp.float32)

        @pl.loop(0, RPT // CHUNK)
        def _(j):
            pltpu.sync_copy(
                rows_v.at[0], acc_sh.at[pl.ds(s * RPT + j * CHUNK, CHUNK)])

        plsc.subcore_barrier()
        base = (c * NS + s) * CPT

        @pl.loop(0, CPT)
        def _(k):
            pltpu.sync_copy(src_hbm.at[base + k], src_v.at[0])
            pltpu.sync_copy(dst_hbm.at[base + k], dst_v.at[0])
            pltpu.async_copy(g_hbm.at[src_v.at[0]], rows_v.at[0], sem).wait()
            pltpu.sync_copy(rows_v.at[0], acc_sh.at[dst_v.at[0]], add=True)

        plsc.subcore_barrier()
        pltpu.sync_copy(acc_sh.at[pl.ds(s * RPT, RPT)],
                        out_hbm.at[c, pl.ds(s * RPT, RPT)])

    return segk(g, srcp, dstp)


_BM = 1000
_GRID = N // _BM


def _mm(x, w):
    k, kout = w.shape

    def body(x_ref, w_ref, o_ref):
        o_ref[...] = jnp.dot(x_ref[...], w_ref[...],
                             preferred_element_type=jnp.float32)

    return pl.pallas_call(
        body,
        grid=(_GRID,),
        in_specs=[pl.BlockSpec((_BM, k), lambda i: (i, 0)),
                  pl.BlockSpec((k, kout), lambda i: (0, 0))],
        out_specs=pl.BlockSpec((_BM, kout), lambda i: (i, 0)),
        out_shape=jax.ShapeDtypeStruct((N, kout), jnp.float32),
    )(x, w)


def _scale_by_dis(degp, h):
    def body(p_ref, h_ref, g_ref, dis_ref):
        dtot = p_ref[0, :, 0:1] + p_ref[1, :, 0:1] + 1.0
        r = lax.rsqrt(dtot)
        g_ref[...] = h_ref[...] * r
        dis_ref[...] = jnp.broadcast_to(r, dis_ref.shape)

    return pl.pallas_call(
        body,
        grid=(_GRID,),
        in_specs=[pl.BlockSpec((NC, _BM, 16), lambda i: (0, i, 0)),
                  pl.BlockSpec((_BM, D_H), lambda i: (i, 0))],
        out_specs=[pl.BlockSpec((_BM, D_H), lambda i: (i, 0)),
                   pl.BlockSpec((_BM, D_H), lambda i: (i, 0))],
        out_shape=[jax.ShapeDtypeStruct((N, D_H), jnp.float32),
                   jax.ShapeDtypeStruct((N, D_H), jnp.float32)],
    )(degp, h)


def _layer1_finish(s1, g1, dis, b1, w2):
    def body(p_ref, g1_ref, dis_ref, b1_ref, w2_ref, g2_ref):
        t = p_ref[0] + p_ref[1] + g1_ref[...]
        t = jnp.maximum(dis_ref[...] * t + b1_ref[...], 0.0)
        h2 = jnp.dot(t, w2_ref[...], preferred_element_type=jnp.float32)
        g2_ref[...] = dis_ref[:, :D_EMB] * h2

    return pl.pallas_call(
        body,
        grid=(_GRID,),
        in_specs=[pl.BlockSpec((NC, _BM, D_H), lambda i: (0, i, 0)),
                  pl.BlockSpec((_BM, D_H), lambda i: (i, 0)),
                  pl.BlockSpec((_BM, D_H), lambda i: (i, 0)),
                  pl.BlockSpec((D_H,), lambda i: (0,)),
                  pl.BlockSpec((D_H, D_EMB), lambda i: (0, 0))],
        out_specs=pl.BlockSpec((_BM, D_EMB), lambda i: (i, 0)),
        out_shape=jax.ShapeDtypeStruct((N, D_EMB), jnp.float32),
    )(s1, g1, dis, b1, w2)


def _layer2_finish(s2, g2, dis, b2, wl, bl):
    def body(p_ref, g2_ref, dis_ref, b2_ref, wl_ref, bl_ref, o_ref):
        e = (dis_ref[:, :D_EMB] * (p_ref[0] + p_ref[1] + g2_ref[...])
             + b2_ref[...])
        logits = jnp.dot(e, wl_ref[...],
                         preferred_element_type=jnp.float32) + bl_ref[...]
        m = jnp.max(logits, axis=-1, keepdims=True)
        z = logits - m
        o_ref[...] = z - jnp.log(jnp.sum(jnp.exp(z), axis=-1, keepdims=True))

    return pl.pallas_call(
        body,
        grid=(_GRID,),
        in_specs=[pl.BlockSpec((NC, _BM, D_EMB), lambda i: (0, i, 0)),
                  pl.BlockSpec((_BM, D_EMB), lambda i: (i, 0)),
                  pl.BlockSpec((_BM, D_H), lambda i: (i, 0)),
                  pl.BlockSpec((D_EMB,), lambda i: (0,)),
                  pl.BlockSpec((D_EMB, D_OUT), lambda i: (0, 0)),
                  pl.BlockSpec((D_OUT,), lambda i: (0,))],
        out_specs=pl.BlockSpec((_BM, D_OUT), lambda i: (i, 0)),
        out_shape=jax.ShapeDtypeStruct((N, D_OUT), jnp.float32),
    )(s2, g2, dis, b2, wl, bl)


def _pad_edges(edge_index):
    src = edge_index[0]
    dst = edge_index[1]
    pad = EP - E
    ar = jnp.arange(pad, dtype=jnp.int32)
    psrc = ar % N
    pdst = N + ar % (NACC - N)
    srcp = jnp.concatenate([src, psrc]).reshape(NCH, CHUNK)
    dstp = jnp.concatenate([dst, pdst]).reshape(NCH, CHUNK)
    return srcp, dstp


def kernel(x, edge_index, W1, b1, W2, b2, Wl, bl):
    srcp, dstp = _pad_edges(edge_index)
    degp = _deg_partials(dstp)
    h1x = _mm(x, W1)
    g1, dis = _scale_by_dis(degp, h1x)
    s1 = _segsum_partials(g1, srcp, dstp, D_H)
    g2 = _layer1_finish(s1, g1, dis, b1, W2)
    s2 = _segsum_partials(g2, srcp, dstp, D_EMB)
    return _layer2_finish(s2, g2, dis, b2, Wl, bl)


# speedup vs baseline: 34.3337x; 1.9894x over previous
import functools

import jax
import jax.numpy as jnp
from jax import lax
from jax.experimental import pallas as pl
from jax.experimental.pallas import tpu as pltpu
from jax.experimental.pallas import tpu_sc as plsc

N = 10000
E = 320000
D_IN = 128
D_H = 128
D_EMB = 64
D_OUT = 40

NC = 2
NS = 16
NW = NC * NS
CHUNK = 128
CPT = 80
EP = NW * CPT * CHUNK
NCH = EP // CHUNK
NACC = 10240
RPT = NACC // NS

_MESH = plsc.VectorSubcoreMesh(core_axis_name="c", subcore_axis_name="s")


def _deg_partials(dstp):
    @functools.partial(
        pl.kernel,
        out_type=jax.ShapeDtypeStruct((NC, NACC, 16), jnp.float32),
        mesh=_MESH,
        compiler_params=pltpu.CompilerParams(use_tc_tiling_on_sc=False),
        scratch_types=[
            pltpu.VMEM((CPT, CHUNK), jnp.int32),
            pltpu.VMEM((CHUNK, 16), jnp.float32),
            pltpu.VMEM((CHUNK, 16), jnp.float32),
            pltpu.VMEM_SHARED((NACC, 16), jnp.float32),
        ],
    )
    def degk(dst_hbm, out_hbm, dst_v, ones_v, zeros_v, acc_sh):
        c = lax.axis_index("c")
        s = lax.axis_index("s")

        @pl.loop(0, CHUNK)
        def _(r):
            ones_v[r, pl.ds(0, 16)] = jnp.ones((16,), jnp.float32)
            zeros_v[r, pl.ds(0, 16)] = jnp.zeros((16,), jnp.float32)

        @pl.loop(0, RPT // CHUNK)
        def _(j):
            pltpu.sync_copy(
                zeros_v, acc_sh.at[pl.ds(s * RPT + j * CHUNK, CHUNK)])

        plsc.subcore_barrier()
        base = (c * NS + s) * CPT
        pltpu.sync_copy(dst_hbm.at[pl.ds(base, CPT)], dst_v)

        @pl.loop(0, CPT)
        def _(k):
            pltpu.sync_copy(ones_v, acc_sh.at[dst_v.at[k]], add=True)

        plsc.subcore_barrier()
        pltpu.sync_copy(acc_sh.at[pl.ds(s * RPT, RPT)],
                        out_hbm.at[c, pl.ds(s * RPT, RPT)])

    return degk(dstp)


def _segsum_partials(g, srcp, dstp, d):
    @functools.partial(
        pl.kernel,
        out_type=jax.ShapeDtypeStruct((NC, NACC, d), jnp.float32),
        mesh=_MESH,
        compiler_params=pltpu.CompilerParams(use_tc_tiling_on_sc=(d == 128)),
        scratch_types=[
            pltpu.VMEM((CPT // 2, CHUNK), jnp.int32),
            pltpu.VMEM((CPT // 2, CHUNK), jnp.int32),
            pltpu.VMEM((2, CHUNK, d), jnp.float32),
            pltpu.VMEM_SHARED((NACC, d), jnp.float32),
            pltpu.SemaphoreType.DMA,
            pltpu.SemaphoreType.DMA,
        ],
    )
    def segk(g_hbm, src_hbm, dst_hbm, out_hbm, src_v, dst_v, rows_v, acc_sh,
             sem0, sem1):
        c = lax.axis_index("c")
        s = lax.axis_index("s")
        half = CPT // 2

        @pl.loop(0, CHUNK)
        def _(r):
            for j in range(d // 16):
                rows_v[0, r, pl.ds(j * 16, 16)] = jnp.zeros((16,), jnp.float32)

        @pl.loop(0, RPT // CHUNK)
        def _(j):
            pltpu.sync_copy(
                rows_v.at[0], acc_sh.at[pl.ds(s * RPT + j * CHUNK, CHUNK)])

        plsc.subcore_barrier()
        base = (c * NS + s) * CPT

        for phase in range(2):
            pltpu.sync_copy(
                src_hbm.at[pl.ds(base + phase * half, half)], src_v)
            pltpu.sync_copy(
                dst_hbm.at[pl.ds(base + phase * half, half)], dst_v)
            pltpu.async_copy(g_hbm.at[src_v.at[0]], rows_v.at[0], sem0)

            @pl.loop(0, half // 2)
            def _(t):
                k = 2 * t
                pltpu.async_copy(
                    g_hbm.at[src_v.at[k + 1]], rows_v.at[1], sem1)
                pltpu.make_async_copy(
                    g_hbm.at[src_v.at[k]], rows_v.at[0], sem0).wait()
                pltpu.sync_copy(
                    rows_v.at[0], acc_sh.at[dst_v.at[k]], add=True)

                @pl.when(t < half // 2 - 1)
                def _():
                    pltpu.async_copy(
                        g_hbm.at[src_v.at[k + 2]], rows_v.at[0], sem0)

                pltpu.make_async_copy(
                    g_hbm.at[src_v.at[k + 1]], rows_v.at[1], sem1).wait()
                pltpu.sync_copy(
                    rows_v.at[1], acc_sh.at[dst_v.at[k + 1]], add=True)

        plsc.subcore_barrier()
        pltpu.sync_copy(acc_sh.at[pl.ds(s * RPT, RPT)],
                        out_hbm.at[c, pl.ds(s * RPT, RPT)])

    return segk(g, srcp, dstp)


_BM = 1000
_GRID = N // _BM


def _mm(x, w):
    k, kout = w.shape

    def body(x_ref, w_ref, o_ref):
        o_ref[...] = jnp.dot(x_ref[...], w_ref[...],
                             preferred_element_type=jnp.float32)

    return pl.pallas_call(
        body,
        grid=(_GRID,),
        in_specs=[pl.BlockSpec((_BM, k), lambda i: (i, 0)),
                  pl.BlockSpec((k, kout), lambda i: (0, 0))],
        out_specs=pl.BlockSpec((_BM, kout), lambda i: (i, 0)),
        out_shape=jax.ShapeDtypeStruct((N, kout), jnp.float32),
    )(x, w)


def _scale_by_dis(degp, h):
    def body(p_ref, h_ref, g_ref, dis_ref):
        dtot = p_ref[0, :, 0:1] + p_ref[1, :, 0:1] + 1.0
        r = lax.rsqrt(dtot)
        g_ref[...] = h_ref[...] * r
        dis_ref[...] = jnp.broadcast_to(r, dis_ref.shape)

    return pl.pallas_call(
        body,
        grid=(_GRID,),
        in_specs=[pl.BlockSpec((NC, _BM, 16), lambda i: (0, i, 0)),
                  pl.BlockSpec((_BM, D_H), lambda i: (i, 0))],
        out_specs=[pl.BlockSpec((_BM, D_H), lambda i: (i, 0)),
                   pl.BlockSpec((_BM, D_H), lambda i: (i, 0))],
        out_shape=[jax.ShapeDtypeStruct((N, D_H), jnp.float32),
                   jax.ShapeDtypeStruct((N, D_H), jnp.float32)],
    )(degp, h)


def _layer1_finish(s1, g1, dis, b1, w2):
    def body(p_ref, g1_ref, dis_ref, b1_ref, w2_ref, g2_ref):
        t = p_ref[0] + p_ref[1] + g1_ref[...]
        t = jnp.maximum(dis_ref[...] * t + b1_ref[...], 0.0)
        h2 = jnp.dot(t, w2_ref[...], preferred_element_type=jnp.float32)
        g2_ref[...] = dis_ref[:, :D_EMB] * h2

    return pl.pallas_call(
        body,
        grid=(_GRID,),
        in_specs=[pl.BlockSpec((NC, _BM, D_H), lambda i: (0, i, 0)),
                  pl.BlockSpec((_BM, D_H), lambda i: (i, 0)),
                  pl.BlockSpec((_BM, D_H), lambda i: (i, 0)),
                  pl.BlockSpec((D_H,), lambda i: (0,)),
                  pl.BlockSpec((D_H, D_EMB), lambda i: (0, 0))],
        out_specs=pl.BlockSpec((_BM, D_EMB), lambda i: (i, 0)),
        out_shape=jax.ShapeDtypeStruct((N, D_EMB), jnp.float32),
    )(s1, g1, dis, b1, w2)


def _layer2_finish(s2, g2, dis, b2, wl, bl):
    def body(p_ref, g2_ref, dis_ref, b2_ref, wl_ref, bl_ref, o_ref):
        e = (dis_ref[:, :D_EMB] * (p_ref[0] + p_ref[1] + g2_ref[...])
             + b2_ref[...])
        logits = jnp.dot(e, wl_ref[...],
                         preferred_element_type=jnp.float32) + bl_ref[...]
        m = jnp.max(logits, axis=-1, keepdims=True)
        z = logits - m
        o_ref[...] = z - jnp.log(jnp.sum(jnp.exp(z), axis=-1, keepdims=True))

    return pl.pallas_call(
        body,
        grid=(_GRID,),
        in_specs=[pl.BlockSpec((NC, _BM, D_EMB), lambda i: (0, i, 0)),
                  pl.BlockSpec((_BM, D_EMB), lambda i: (i, 0)),
                  pl.BlockSpec((_BM, D_H), lambda i: (i, 0)),
                  pl.BlockSpec((D_EMB,), lambda i: (0,)),
                  pl.BlockSpec((D_EMB, D_OUT), lambda i: (0, 0)),
                  pl.BlockSpec((D_OUT,), lambda i: (0,))],
        out_specs=pl.BlockSpec((_BM, D_OUT), lambda i: (i, 0)),
        out_shape=jax.ShapeDtypeStruct((N, D_OUT), jnp.float32),
    )(s2, g2, dis, b2, wl, bl)


def _pad_edges(edge_index):
    src = edge_index[0]
    dst = edge_index[1]
    pad = EP - E
    ar = jnp.arange(pad, dtype=jnp.int32)
    psrc = ar % N
    pdst = N + ar % (NACC - N)
    srcp = jnp.concatenate([src, psrc]).reshape(NCH, CHUNK)
    dstp = jnp.concatenate([dst, pdst]).reshape(NCH, CHUNK)
    return srcp, dstp


def kernel(x, edge_index, W1, b1, W2, b2, Wl, bl):
    srcp, dstp = _pad_edges(edge_index)
    degp = _deg_partials(dstp)
    h1x = _mm(x, W1)
    g1, dis = _scale_by_dis(degp, h1x)
    s1 = _segsum_partials(g1, srcp, dstp, D_H)
    g2 = _layer1_finish(s1, g1, dis, b1, W2)
    s2 = _segsum_partials(g2, srcp, dstp, D_EMB)
    return _layer2_finish(s2, g2, dis, b2, Wl, bl)


# speedup vs baseline: 37.5428x; 1.0935x over previous
import functools

import jax
import jax.numpy as jnp
from jax import lax
from jax.experimental import pallas as pl
from jax.experimental.pallas import tpu as pltpu
from jax.experimental.pallas import tpu_sc as plsc

N = 10000
E = 320000
D_IN = 128
D_H = 128
D_EMB = 64
D_OUT = 40

NC = 2
NS = 16
NW = NC * NS
CHUNK = 128
CPT = 80
EP = NW * CPT * CHUNK
NCH = EP // CHUNK
NACC = 10240
RPT = NACC // NS

_MESH = plsc.VectorSubcoreMesh(core_axis_name="c", subcore_axis_name="s")


def _deg_partials(dstp):
    @functools.partial(
        pl.kernel,
        out_type=jax.ShapeDtypeStruct((NC, NACC, 16), jnp.float32),
        mesh=_MESH,
        compiler_params=pltpu.CompilerParams(use_tc_tiling_on_sc=False),
        scratch_types=[
            pltpu.VMEM((CPT, CHUNK), jnp.int32),
            pltpu.VMEM((CHUNK, 16), jnp.float32),
            pltpu.VMEM((CHUNK, 16), jnp.float32),
            pltpu.VMEM_SHARED((NACC, 16), jnp.float32),
        ],
    )
    def degk(dst_hbm, out_hbm, dst_v, ones_v, zeros_v, acc_sh):
        c = lax.axis_index("c")
        s = lax.axis_index("s")

        @pl.loop(0, CHUNK)
        def _(r):
            ones_v[r, pl.ds(0, 16)] = jnp.ones((16,), jnp.float32)
            zeros_v[r, pl.ds(0, 16)] = jnp.zeros((16,), jnp.float32)

        @pl.loop(0, RPT // CHUNK)
        def _(j):
            pltpu.sync_copy(
                zeros_v, acc_sh.at[pl.ds(s * RPT + j * CHUNK, CHUNK)])

        plsc.subcore_barrier()
        base = (c * NS + s) * CPT
        pltpu.sync_copy(dst_hbm.at[pl.ds(base, CPT)], dst_v)

        @pl.loop(0, CPT)
        def _(k):
            pltpu.sync_copy(ones_v, acc_sh.at[dst_v.at[k]], add=True)

        plsc.subcore_barrier()
        pltpu.sync_copy(acc_sh.at[pl.ds(s * RPT, RPT)],
                        out_hbm.at[c, pl.ds(s * RPT, RPT)])

    return degk(dstp)


def _segsum_partials(g, srcp, dstp, d):
    @functools.partial(
        pl.kernel,
        out_type=jax.ShapeDtypeStruct((NC, NACC, d), jnp.bfloat16),
        mesh=_MESH,
        compiler_params=pltpu.CompilerParams(use_tc_tiling_on_sc=False),
        scratch_types=[
            pltpu.VMEM((CPT // 2, CHUNK), jnp.int32),
            pltpu.VMEM((CPT // 2, CHUNK), jnp.int32),
            pltpu.VMEM((2, CHUNK, d), jnp.bfloat16),
            pltpu.VMEM_SHARED((NACC, d), jnp.bfloat16),
            pltpu.SemaphoreType.DMA,
            pltpu.SemaphoreType.DMA,
        ],
    )
    def segk(g_hbm, src_hbm, dst_hbm, out_hbm, src_v, dst_v, rows_v, acc_sh,
             sem0, sem1):
        c = lax.axis_index("c")
        s = lax.axis_index("s")
        half = CPT // 2

        @pl.loop(0, CHUNK)
        def _(r):
            for j in range(d // 32):
                rows_v[0, r, pl.ds(j * 32, 32)] = jnp.zeros((32,),
                                                            jnp.bfloat16)

        @pl.loop(0, RPT // CHUNK)
        def _(j):
            pltpu.sync_copy(
                rows_v.at[0], acc_sh.at[pl.ds(s * RPT + j * CHUNK, CHUNK)])

        plsc.subcore_barrier()
        base = (c * NS + s) * CPT

        for phase in range(2):
            pltpu.sync_copy(
                src_hbm.at[pl.ds(base + phase * half, half)], src_v)
            pltpu.sync_copy(
                dst_hbm.at[pl.ds(base + phase * half, half)], dst_v)
            pltpu.async_copy(g_hbm.at[src_v.at[0]], rows_v.at[0], sem0)

            @pl.loop(0, half // 2)
            def _(t):
                k = 2 * t
                pltpu.async_copy(
                    g_hbm.at[src_v.at[k + 1]], rows_v.at[1], sem1)
                pltpu.make_async_copy(
                    g_hbm.at[src_v.at[k]], rows_v.at[0], sem0).wait()
                pltpu.sync_copy(
                    rows_v.at[0], acc_sh.at[dst_v.at[k]], add=True)

                @pl.when(t < half // 2 - 1)
                def _():
                    pltpu.async_copy(
                        g_hbm.at[src_v.at[k + 2]], rows_v.at[0], sem0)

                pltpu.make_async_copy(
                    g_hbm.at[src_v.at[k + 1]], rows_v.at[1], sem1).wait()
                pltpu.sync_copy(
                    rows_v.at[1], acc_sh.at[dst_v.at[k + 1]], add=True)

        plsc.subcore_barrier()
        pltpu.sync_copy(acc_sh.at[pl.ds(s * RPT, RPT)],
                        out_hbm.at[c, pl.ds(s * RPT, RPT)])

    return segk(g, srcp, dstp)


_BM = 1000
_GRID = N // _BM


def _mm(x, w):
    k, kout = w.shape

    def body(x_ref, w_ref, o_ref):
        o_ref[...] = jnp.dot(x_ref[...], w_ref[...],
                             preferred_element_type=jnp.float32)

    return pl.pallas_call(
        body,
        grid=(_GRID,),
        in_specs=[pl.BlockSpec((_BM, k), lambda i: (i, 0)),
                  pl.BlockSpec((k, kout), lambda i: (0, 0))],
        out_specs=pl.BlockSpec((_BM, kout), lambda i: (i, 0)),
        out_shape=jax.ShapeDtypeStruct((N, kout), jnp.float32),
    )(x, w)


def _scale_by_dis(degp, h):
    def body(p_ref, h_ref, g_ref, dis_ref):
        dtot = p_ref[0, :, 0:1] + p_ref[1, :, 0:1] + 1.0
        r = lax.rsqrt(dtot)
        g_ref[...] = (h_ref[...] * r).astype(jnp.bfloat16)
        dis_ref[...] = jnp.broadcast_to(r, dis_ref.shape)

    return pl.pallas_call(
        body,
        grid=(_GRID,),
        in_specs=[pl.BlockSpec((NC, _BM, 16), lambda i: (0, i, 0)),
                  pl.BlockSpec((_BM, D_H), lambda i: (i, 0))],
        out_specs=[pl.BlockSpec((_BM, D_H), lambda i: (i, 0)),
                   pl.BlockSpec((_BM, D_H), lambda i: (i, 0))],
        out_shape=[jax.ShapeDtypeStruct((N, D_H), jnp.bfloat16),
                   jax.ShapeDtypeStruct((N, D_H), jnp.float32)],
    )(degp, h)


def _layer1_finish(s1, g1, dis, b1, w2):
    def body(p_ref, g1_ref, dis_ref, b1_ref, w2_ref, g2_ref):
        t = (p_ref[0].astype(jnp.float32) + p_ref[1].astype(jnp.float32)
             + g1_ref[...].astype(jnp.float32))
        t = jnp.maximum(dis_ref[...] * t + b1_ref[...], 0.0)
        h2 = jnp.dot(t, w2_ref[...], preferred_element_type=jnp.float32)
        g2_ref[...] = (dis_ref[:, :D_EMB] * h2).astype(jnp.bfloat16)

    return pl.pallas_call(
        body,
        grid=(_GRID,),
        in_specs=[pl.BlockSpec((NC, _BM, D_H), lambda i: (0, i, 0)),
                  pl.BlockSpec((_BM, D_H), lambda i: (i, 0)),
                  pl.BlockSpec((_BM, D_H), lambda i: (i, 0)),
                  pl.BlockSpec((D_H,), lambda i: (0,)),
                  pl.BlockSpec((D_H, D_EMB), lambda i: (0, 0))],
        out_specs=pl.BlockSpec((_BM, D_EMB), lambda i: (i, 0)),
        out_shape=jax.ShapeDtypeStruct((N, D_EMB), jnp.bfloat16),
    )(s1, g1, dis, b1, w2)


def _layer2_finish(s2, g2, dis, b2, wl, bl):
    def body(p_ref, g2_ref, dis_ref, b2_ref, wl_ref, bl_ref, o_ref):
        acc = (p_ref[0].astype(jnp.float32) + p_ref[1].astype(jnp.float32)
               + g2_ref[...].astype(jnp.float32))
        e = dis_ref[:, :D_EMB] * acc + b2_ref[...]
        logits = jnp.dot(e, wl_ref[...],
                         preferred_element_type=jnp.float32) + bl_ref[...]
        m = jnp.max(logits, axis=-1, keepdims=True)
        z = logits - m
        o_ref[...] = z - jnp.log(jnp.sum(jnp.exp(z), axis=-1, keepdims=True))

    return pl.pallas_call(
        body,
        grid=(_GRID,),
        in_specs=[pl.BlockSpec((NC, _BM, D_EMB), lambda i: (0, i, 0)),
                  pl.BlockSpec((_BM, D_EMB), lambda i: (i, 0)),
                  pl.BlockSpec((_BM, D_H), lambda i: (i, 0)),
                  pl.BlockSpec((D_EMB,), lambda i: (0,)),
                  pl.BlockSpec((D_EMB, D_OUT), lambda i: (0, 0)),
                  pl.BlockSpec((D_OUT,), lambda i: (0,))],
        out_specs=pl.BlockSpec((_BM, D_OUT), lambda i: (i, 0)),
        out_shape=jax.ShapeDtypeStruct((N, D_OUT), jnp.float32),
    )(s2, g2, dis, b2, wl, bl)


def _pad_edges(edge_index):
    src = edge_index[0]
    dst = edge_index[1]
    pad = EP - E
    ar = jnp.arange(pad, dtype=jnp.int32)
    psrc = ar % N
    pdst = N + ar % (NACC - N)
    srcp = jnp.concatenate([src, psrc]).reshape(NCH, CHUNK)
    dstp = jnp.concatenate([dst, pdst]).reshape(NCH, CHUNK)
    return srcp, dstp


def kernel(x, edge_index, W1, b1, W2, b2, Wl, bl):
    srcp, dstp = _pad_edges(edge_index)
    degp = _deg_partials(dstp)
    h1x = _mm(x, W1)
    g1, dis = _scale_by_dis(degp, h1x)
    s1 = _segsum_partials(g1, srcp, dstp, D_H)
    g2 = _layer1_finish(s1, g1, dis, b1, W2)
    s2 = _segsum_partials(g2, srcp, dstp, D_EMB)
    return _layer2_finish(s2, g2, dis, b2, Wl, bl)
